# baseline (speedup 1.0000x reference)
import functools

import jax
import jax.numpy as jnp
from jax.experimental import pallas as pl
from jax.experimental.pallas import tpu as pltpu
from jax.experimental.pallas import tpu_sc as plsc

_T = 2048
_D = 768
_FF = 3072
_E = 8
_K = 2
_BTR = 256
_BT = 128
_P = _K * _T + _E * _BT
_G = _P // _BT
_GW = 128


def _gelu_exact(v):
    return 0.5 * v * (1.0 + jax.lax.erf(v * (2.0 ** -0.5)))


def _router_body(logits_ref, idx_ref, w_ref, usage_ref):
    g = pl.program_id(0)
    logits = logits_ref[...]
    cols = jax.lax.broadcasted_iota(jnp.int32, (_BTR, _E), 1)
    m1 = jnp.max(logits, axis=-1, keepdims=True)
    a1 = jnp.min(jnp.where(logits == m1, cols, _E), axis=-1, keepdims=True)
    rest = jnp.where(cols == a1, -jnp.inf, logits)
    m2 = jnp.max(rest, axis=-1, keepdims=True)
    a2 = jnp.min(jnp.where(rest == m2, cols, _E), axis=-1, keepdims=True)
    e2 = jnp.exp(m2 - m1)
    w1 = 1.0 / (1.0 + e2)
    w2 = e2 * w1
    idx_ref[...] = jnp.concatenate([a1, a2], axis=1)
    w_ref[...] = jnp.concatenate([w1, w2], axis=1)
    ex = jnp.exp(logits - m1)
    p = ex / jnp.sum(ex, axis=-1, keepdims=True)

    @pl.when(g == 0)
    def _():
        usage_ref[...] = jnp.zeros_like(usage_ref)

    usage_ref[...] += jnp.sum(p, axis=0, keepdims=True)


def _router(logits):
    grid = (_T // _BTR,)
    return pl.pallas_call(
        _router_body,
        grid=grid,
        in_specs=[
            pl.BlockSpec((_BTR, _E), lambda g: (g, 0)),
        ],
        out_specs=[
            pl.BlockSpec((_BTR, _K), lambda g: (g, 0)),
            pl.BlockSpec((_BTR, _K), lambda g: (g, 0)),
            pl.BlockSpec((1, _E), lambda g: (0, 0)),
        ],
        out_shape=[
            jax.ShapeDtypeStruct((_T, _K), jnp.int32),
            jax.ShapeDtypeStruct((_T, _K), jnp.float32),
            jax.ShapeDtypeStruct((1, _E), jnp.float32),
        ],
        compiler_params=pltpu.CompilerParams(
            dimension_semantics=("arbitrary",)),
    )(logits)


def _routing_setup(top2i):
    e_flat = top2i.reshape(-1)
    oh = (e_flat[:, None] == jnp.arange(_E)[None, :]).astype(jnp.int32)
    counts = jnp.sum(oh, axis=0)
    ranks = jnp.cumsum(oh, axis=0) - oh
    rank = jnp.sum(ranks * oh, axis=1)
    padded = ((counts + _BT - 1) // _BT) * _BT
    ends = jnp.cumsum(padded)
    starts = ends - padded
    pos = (starts[e_flat] + rank).astype(jnp.int32)
    gstart = jnp.arange(_G, dtype=jnp.int32) * _BT
    block_expert = jnp.clip(
        jnp.searchsorted(ends, gstart, side="right"), 0, _E - 1
    ).astype(jnp.int32)
    block_active = (gstart < ends[-1]).astype(jnp.int32)
    return pos, block_expert, block_active


def _sc_mesh():
    return plsc.VectorSubcoreMesh(core_axis_name="core",
                                  subcore_axis_name="subcore")


def _dispatch(x_pk, src, dst):
    m = src.shape[1]
    d = x_pk.shape[1]

    @pl.kernel(out_type=jax.ShapeDtypeStruct((_P, d), x_pk.dtype),
               mesh=_sc_mesh(),
               scratch_types=[pltpu.VMEM((_GW, d), x_pk.dtype)])
    def k(x_hbm, src_hbm, dst_hbm, o_hbm, stage):
        def body(src_vmem, dst_vmem):
            pltpu.sync_copy(x_hbm.at[src_vmem.at[0]], stage)
            pltpu.sync_copy(stage, o_hbm.at[dst_vmem.at[0]])

        pltpu.emit_pipeline(
            body,
            grid=(m // _GW,),
            in_specs=[pl.BlockSpec((1, _GW), index_map=lambda i: (0, i)),
                      pl.BlockSpec((1, _GW), index_map=lambda i: (0, i))],
            out_specs=[],
            core_axis_name=("core", "subcore"),
            dimension_semantics=(pltpu.PARALLEL,),
        )(src_hbm, dst_hbm)

    return k(x_pk, src, dst)


def _gather_rows(data, idx):
    m = idx.shape[1]
    d = data.shape[1]

    @pl.kernel(out_type=jax.ShapeDtypeStruct((m, d), data.dtype),
               mesh=_sc_mesh())
    def k(x_hbm, i_hbm, o_hbm):
        def body(i_vmem, o_vmem):
            pltpu.sync_copy(x_hbm.at[i_vmem.at[0]], o_vmem)

        pltpu.emit_pipeline(
            body,
            grid=(m // _GW,),
            in_specs=[pl.BlockSpec((1, _GW), index_map=lambda i: (0, i))],
            out_specs=[pl.BlockSpec((_GW, d), index_map=lambda i: (i, 0))],
            core_axis_name=("core", "subcore"),
            dimension_semantics=(pltpu.PARALLEL,),
        )(i_hbm, o_hbm)

    return k(data, idx)


def _gmm_body(be_ref, act_ref, x_ref, w1_ref, b1_ref, w2_ref, b2_ref,
              o_ref):
    g = pl.program_id(0)

    @pl.when(act_ref[g] == 1)
    def _():
        x = x_ref[...].astype(jnp.bfloat16)
        h = jnp.dot(x, w1_ref[0].astype(jnp.bfloat16),
                    preferred_element_type=jnp.float32)
        h = _gelu_exact(h + b1_ref[0])
        y = jnp.dot(h.astype(jnp.bfloat16), w2_ref[0].astype(jnp.bfloat16),
                    preferred_element_type=jnp.float32)
        y = y + b2_ref[0]
        hd = _D // 2
        o_ref[0] = y[:, :hd]
        o_ref[1] = y[:, hd:]


def _gmm(x_sorted, block_expert, block_active, W1, b1, W2, b2):
    grid_spec = pltpu.PrefetchScalarGridSpec(
        num_scalar_prefetch=2,
        grid=(_G,),
        in_specs=[
            pl.BlockSpec((_BT, _D), lambda g, be, act: (g, 0)),
            pl.BlockSpec((1, _D, _FF), lambda g, be, act: (be[g], 0, 0)),
            pl.BlockSpec((1, 1, _FF), lambda g, be, act: (be[g], 0, 0)),
            pl.BlockSpec((1, _FF, _D), lambda g, be, act: (be[g], 0, 0)),
            pl.BlockSpec((1, 1, _D), lambda g, be, act: (be[g], 0, 0)),
        ],
        out_specs=pl.BlockSpec((2, _BT, _D // 2), lambda g, be, act: (0, g, 0)),
    )
    return pl.pallas_call(
        _gmm_body,
        grid_spec=grid_spec,
        out_shape=jax.ShapeDtypeStruct((2, _P, _D // 2), jnp.float32),
        compiler_params=pltpu.CompilerParams(
            dimension_semantics=("parallel",)),
    )(block_expert, block_active, x_sorted,
      W1, b1.reshape(_E, 1, _FF), W2, b2.reshape(_E, 1, _D))


def _combine_body(a0_ref, a1_ref, a2_ref, a3_ref, w_ref, o_ref):
    hd = _D // 2
    w0 = w_ref[:, 0:1]
    w1 = w_ref[:, 1:2]
    o_ref[:, :hd] = w0 * a0_ref[0] + w1 * a2_ref[0]
    o_ref[:, hd:] = w0 * a1_ref[0] + w1 * a3_ref[0]


def _combine(pairs4, top2w):
    grid = (_T // _BTR,)
    plane = lambda j: pl.BlockSpec((1, _BTR, _D // 2),
                                   lambda g, j=j: (j, g, 0))
    return pl.pallas_call(
        _combine_body,
        grid=grid,
        in_specs=[plane(0), plane(1), plane(2), plane(3),
                  pl.BlockSpec((_BTR, _K), lambda g: (g, 0))],
        out_specs=pl.BlockSpec((_BTR, _D), lambda g: (g, 0)),
        out_shape=jax.ShapeDtypeStruct((_T, _D), jnp.float32),
        compiler_params=pltpu.CompilerParams(
            dimension_semantics=("parallel",)),
    )(pairs4, pairs4, pairs4, pairs4, top2w)


def kernel(x, router_w1, router_b1, router_w2, router_b2, W1, b1, W2, b2):
    batch, seq, d_model = x.shape
    x_flat = x.reshape(-1, d_model)

    h = jax.nn.gelu(x_flat @ router_w1 + router_b1, approximate=False)
    logits = h @ router_w2 + router_b2

    top2i, top2w, usage_sum = _router(logits)
    usage = usage_sum[0] / jnp.float32(_T)
    lb_loss = 0.01 * jnp.sum((usage - jnp.mean(usage)) ** 2) / (_E - 1)

    pos, block_expert, block_active = _routing_setup(top2i)
    src = (jnp.arange(_K * _T, dtype=jnp.int32) // _K).reshape(1, -1)

    x_sorted = _dispatch(x_flat, src, pos.reshape(1, -1))
    y = _gmm(x_sorted, block_expert, block_active, W1, b1, W2, b2)
    posr = pos.reshape(_T, _K)
    idx4 = jnp.concatenate(
        [posr[:, 0], _P + posr[:, 0], posr[:, 1], _P + posr[:, 1]]
    ).reshape(1, -1)
    pairs4 = _gather_rows(y.reshape(2 * _P, _D // 2), idx4)
    out = _combine(pairs4.reshape(4, _T, _D // 2), top2w)
    return out.reshape(batch, seq, d_model), lb_loss


# speedup vs baseline: 1.3160x; 1.3160x over previous
import functools

import jax
import jax.numpy as jnp
from jax.experimental import pallas as pl
from jax.experimental.pallas import tpu as pltpu
from jax.experimental.pallas import tpu_sc as plsc

_T = 2048
_D = 768
_FF = 3072
_E = 8
_K = 2
_BTR = 256
_BT = 512
_P = _K * _T + _E * _BT
_G = _P // _BT
_GW = 128


def _gelu_exact(v):
    return 0.5 * v * (1.0 + jax.lax.erf(v * (2.0 ** -0.5)))


def _router_body(logits_ref, idx_ref, w_ref, usage_ref):
    g = pl.program_id(0)
    logits = logits_ref[...]
    cols = jax.lax.broadcasted_iota(jnp.int32, (_BTR, _E), 1)
    m1 = jnp.max(logits, axis=-1, keepdims=True)
    a1 = jnp.min(jnp.where(logits == m1, cols, _E), axis=-1, keepdims=True)
    rest = jnp.where(cols == a1, -jnp.inf, logits)
    m2 = jnp.max(rest, axis=-1, keepdims=True)
    a2 = jnp.min(jnp.where(rest == m2, cols, _E), axis=-1, keepdims=True)
    e2 = jnp.exp(m2 - m1)
    w1 = 1.0 / (1.0 + e2)
    w2 = e2 * w1
    idx_ref[...] = jnp.concatenate([a1, a2], axis=1)
    w_ref[...] = jnp.concatenate([w1, w2], axis=1)
    ex = jnp.exp(logits - m1)
    p = ex / jnp.sum(ex, axis=-1, keepdims=True)

    @pl.when(g == 0)
    def _():
        usage_ref[...] = jnp.zeros_like(usage_ref)

    usage_ref[...] += jnp.sum(p, axis=0, keepdims=True)


def _router(logits):
    grid = (_T // _BTR,)
    return pl.pallas_call(
        _router_body,
        grid=grid,
        in_specs=[
            pl.BlockSpec((_BTR, _E), lambda g: (g, 0)),
        ],
        out_specs=[
            pl.BlockSpec((_BTR, _K), lambda g: (g, 0)),
            pl.BlockSpec((_BTR, _K), lambda g: (g, 0)),
            pl.BlockSpec((1, _E), lambda g: (0, 0)),
        ],
        out_shape=[
            jax.ShapeDtypeStruct((_T, _K), jnp.int32),
            jax.ShapeDtypeStruct((_T, _K), jnp.float32),
            jax.ShapeDtypeStruct((1, _E), jnp.float32),
        ],
        compiler_params=pltpu.CompilerParams(
            dimension_semantics=("arbitrary",)),
    )(logits)


def _routing_setup(top2i):
    e_flat = top2i.reshape(-1)
    oh = (e_flat[:, None] == jnp.arange(_E)[None, :]).astype(jnp.int32)
    counts = jnp.sum(oh, axis=0)
    ranks = jnp.cumsum(oh, axis=0) - oh
    rank = jnp.sum(ranks * oh, axis=1)
    padded = ((counts + _BT - 1) // _BT) * _BT
    ends = jnp.cumsum(padded)
    starts = ends - padded
    pos = (starts[e_flat] + rank).astype(jnp.int32)
    gstart = jnp.arange(_G, dtype=jnp.int32) * _BT
    block_expert = jnp.clip(
        jnp.searchsorted(ends, gstart, side="right"), 0, _E - 1
    ).astype(jnp.int32)
    block_active = (gstart < ends[-1]).astype(jnp.int32)
    return pos, block_expert, block_active


def _sc_mesh():
    return plsc.VectorSubcoreMesh(core_axis_name="core",
                                  subcore_axis_name="subcore")


def _dispatch(x_pk, src, dst):
    m = src.shape[1]
    d = x_pk.shape[1]

    @pl.kernel(out_type=jax.ShapeDtypeStruct((_P, d), x_pk.dtype),
               mesh=_sc_mesh(),
               scratch_types=[pltpu.VMEM((_GW, d), x_pk.dtype)])
    def k(x_hbm, src_hbm, dst_hbm, o_hbm, stage):
        def body(src_vmem, dst_vmem):
            pltpu.sync_copy(x_hbm.at[src_vmem.at[0]], stage)
            pltpu.sync_copy(stage, o_hbm.at[dst_vmem.at[0]])

        pltpu.emit_pipeline(
            body,
            grid=(m // _GW,),
            in_specs=[pl.BlockSpec((1, _GW), index_map=lambda i: (0, i)),
                      pl.BlockSpec((1, _GW), index_map=lambda i: (0, i))],
            out_specs=[],
            core_axis_name=("core", "subcore"),
            dimension_semantics=(pltpu.PARALLEL,),
        )(src_hbm, dst_hbm)

    return k(x_pk, src, dst)


def _gather_rows(data, idx):
    m = idx.shape[1]
    d = data.shape[1]

    @pl.kernel(out_type=jax.ShapeDtypeStruct((m, d), data.dtype),
               mesh=_sc_mesh())
    def k(x_hbm, i_hbm, o_hbm):
        def body(i_vmem, o_vmem):
            pltpu.sync_copy(x_hbm.at[i_vmem.at[0]], o_vmem)

        pltpu.emit_pipeline(
            body,
            grid=(m // _GW,),
            in_specs=[pl.BlockSpec((1, _GW), index_map=lambda i: (0, i))],
            out_specs=[pl.BlockSpec((_GW, d), index_map=lambda i: (i, 0))],
            core_axis_name=("core", "subcore"),
            dimension_semantics=(pltpu.PARALLEL,),
        )(i_hbm, o_hbm)

    return k(data, idx)


def _gmm_body(be_ref, act_ref, x_ref, w1_ref, b1_ref, w2_ref, b2_ref,
              o_ref):
    g = pl.program_id(0)

    @pl.when(act_ref[g] == 1)
    def _():
        x = x_ref[...].astype(jnp.bfloat16)
        h = jnp.dot(x, w1_ref[0].astype(jnp.bfloat16),
                    preferred_element_type=jnp.float32)
        h = _gelu_exact(h + b1_ref[0])
        y = jnp.dot(h.astype(jnp.bfloat16), w2_ref[0].astype(jnp.bfloat16),
                    preferred_element_type=jnp.float32)
        y = y + b2_ref[0]
        hd = _D // 2
        o_ref[0] = y[:, :hd]
        o_ref[1] = y[:, hd:]


def _gmm(x_sorted, block_expert, block_active, W1, b1, W2, b2):
    grid_spec = pltpu.PrefetchScalarGridSpec(
        num_scalar_prefetch=2,
        grid=(_G,),
        in_specs=[
            pl.BlockSpec((_BT, _D), lambda g, be, act: (g, 0)),
            pl.BlockSpec((1, _D, _FF), lambda g, be, act: (be[g], 0, 0)),
            pl.BlockSpec((1, 1, _FF), lambda g, be, act: (be[g], 0, 0)),
            pl.BlockSpec((1, _FF, _D), lambda g, be, act: (be[g], 0, 0)),
            pl.BlockSpec((1, 1, _D), lambda g, be, act: (be[g], 0, 0)),
        ],
        out_specs=pl.BlockSpec((2, _BT, _D // 2), lambda g, be, act: (0, g, 0)),
    )
    return pl.pallas_call(
        _gmm_body,
        grid_spec=grid_spec,
        out_shape=jax.ShapeDtypeStruct((2, _P, _D // 2), jnp.float32),
        compiler_params=pltpu.CompilerParams(
            dimension_semantics=("parallel",)),
    )(block_expert, block_active, x_sorted,
      W1, b1.reshape(_E, 1, _FF), W2, b2.reshape(_E, 1, _D))


def _combine_body(a0_ref, a1_ref, a2_ref, a3_ref, w_ref, o_ref):
    hd = _D // 2
    w0 = w_ref[:, 0:1]
    w1 = w_ref[:, 1:2]
    o_ref[:, :hd] = w0 * a0_ref[0] + w1 * a2_ref[0]
    o_ref[:, hd:] = w0 * a1_ref[0] + w1 * a3_ref[0]


def _combine(pairs4, top2w):
    grid = (_T // _BTR,)
    plane = lambda j: pl.BlockSpec((1, _BTR, _D // 2),
                                   lambda g, j=j: (j, g, 0))
    return pl.pallas_call(
        _combine_body,
        grid=grid,
        in_specs=[plane(0), plane(1), plane(2), plane(3),
                  pl.BlockSpec((_BTR, _K), lambda g: (g, 0))],
        out_specs=pl.BlockSpec((_BTR, _D), lambda g: (g, 0)),
        out_shape=jax.ShapeDtypeStruct((_T, _D), jnp.float32),
        compiler_params=pltpu.CompilerParams(
            dimension_semantics=("parallel",)),
    )(pairs4, pairs4, pairs4, pairs4, top2w)


def kernel(x, router_w1, router_b1, router_w2, router_b2, W1, b1, W2, b2):
    batch, seq, d_model = x.shape
    x_flat = x.reshape(-1, d_model)

    h = jax.nn.gelu(x_flat @ router_w1 + router_b1, approximate=False)
    logits = h @ router_w2 + router_b2

    top2i, top2w, usage_sum = _router(logits)
    usage = usage_sum[0] / jnp.float32(_T)
    lb_loss = 0.01 * jnp.sum((usage - jnp.mean(usage)) ** 2) / (_E - 1)

    pos, block_expert, block_active = _routing_setup(top2i)
    src = (jnp.arange(_K * _T, dtype=jnp.int32) // _K).reshape(1, -1)

    x_sorted = _dispatch(x_flat, src, pos.reshape(1, -1))
    y = _gmm(x_sorted, block_expert, block_active, W1, b1, W2, b2)
    posr = pos.reshape(_T, _K)
    idx4 = jnp.concatenate(
        [posr[:, 0], _P + posr[:, 0], posr[:, 1], _P + posr[:, 1]]
    ).reshape(1, -1)
    pairs4 = _gather_rows(y.reshape(2 * _P, _D // 2), idx4)
    out = _combine(pairs4.reshape(4, _T, _D // 2), top2w)
    return out.reshape(batch, seq, d_model), lb_loss
